# baseline (speedup 1.0000x reference)
import functools

import jax
import jax.numpy as jnp
from jax import lax
from jax.experimental import pallas as pl
from jax.experimental.pallas import tpu as pltpu
from jax.experimental.pallas import tpu_sc as plsc

_CHUNK = 128
_M_BLK = 1024
_KPAD = 1024
_B_SC = 4096


@functools.cache
def _build_sc(V, D, B_sc):
    info = plsc.get_sparse_core_info()
    NC, NS = info.num_cores, info.num_subcores
    NW = NC * NS
    b_per_w = B_sc // NW
    n_ch = b_per_w // _CHUNK
    mesh = plsc.VectorSubcoreMesh(core_axis_name="c", subcore_axis_name="s")

    @functools.partial(
        pl.kernel,
        mesh=mesh,
        out_type=jax.ShapeDtypeStruct((B_sc, D), jnp.float32),
        scratch_types=[
            pltpu.VMEM((n_ch, _CHUNK), jnp.int32),
            pltpu.VMEM((n_ch, _CHUNK, D), jnp.float32),
            pltpu.SemaphoreType.DMA,
            pltpu.SemaphoreType.DMA,
        ],
    )
    def k(table_hbm, idx_hbm, out_hbm, idx_v, rows_v, gsem, osem):
        wid = lax.axis_index("s") * NC + lax.axis_index("c")
        base = wid * b_per_w
        pltpu.sync_copy(idx_hbm.at[wid], idx_v)
        gathers = []
        for j in range(n_ch):
            gathers.append(
                pltpu.async_copy(table_hbm.at[idx_v.at[j]], rows_v.at[j], gsem)
            )
        outs = []
        for j in range(n_ch):
            gathers[j].wait()
            outs.append(
                pltpu.async_copy(
                    rows_v.at[j], out_hbm.at[pl.ds(base + j * _CHUNK, _CHUNK)], osem
                )
            )
        for o in outs:
            o.wait()

    return k, NW, n_ch


@functools.cache
def _build_tc(B_tc, B, D):
    grid = B_tc // _M_BLK

    def body(idx_ref, tab_ref, out_ref):
        idx = idx_ref[0]
        iota_k = lax.broadcasted_iota(jnp.int32, (_KPAD, _M_BLK), 0)
        oht = (iota_k == idx).astype(jnp.bfloat16)
        res = lax.dot_general(
            oht, tab_ref[...],
            (((0,), (0,)), ((), ())),
            preferred_element_type=jnp.float32,
        )
        out_ref[...] = res[:, :D] + res[:, D:]

    return pl.pallas_call(
        body,
        grid=(grid,),
        in_specs=[
            pl.BlockSpec((1, 1, _M_BLK), lambda i: (i, 0, 0)),
            pl.BlockSpec((_KPAD, 2 * D), lambda i: (0, 0)),
        ],
        out_specs=pl.BlockSpec((_M_BLK, D), lambda i: (i, 0)),
        out_shape=jax.ShapeDtypeStruct((B, D), jnp.float32),
    )


def _split_hi_lo(table):
    bits = lax.bitcast_convert_type(table, jnp.uint32)
    hi = lax.bitcast_convert_type((bits >> 16).astype(jnp.uint16), jnp.bfloat16)
    lo = (table - lax.bitcast_convert_type(bits & jnp.uint32(0xFFFF0000),
                                           jnp.float32)).astype(jnp.bfloat16)
    return hi, lo


def kernel(class_number, embedding):
    V, D = embedding.shape
    B = class_number.shape[0]
    B_tc = B - _B_SC
    idx = class_number.astype(jnp.int32)

    sc_k, NW, n_ch = _build_sc(V, D, _B_SC)
    sc_part = sc_k(embedding, idx[B_tc:].reshape(NW, n_ch, _CHUNK))

    hi, lo = _split_hi_lo(embedding)
    tab2 = jnp.pad(jnp.concatenate([hi, lo], axis=1), ((0, _KPAD - V), (0, 0)))
    tc_full = _build_tc(B_tc, B, D)(idx[:B_tc].reshape(B_tc // _M_BLK, 1, _M_BLK),
                                    tab2)

    return lax.dynamic_update_slice(tc_full, sc_part, (B_tc, 0))


# speedup vs baseline: 1.0045x; 1.0045x over previous
import functools

import jax
import jax.numpy as jnp
from jax import lax
from jax.experimental import pallas as pl
from jax.experimental.pallas import tpu as pltpu
from jax.experimental.pallas import tpu_sc as plsc

_CHUNK = 128
_M_BLK = 1024
_KPAD = 1024
_B_SC = 8192


@functools.cache
def _build_sc(V, D, B_sc):
    info = plsc.get_sparse_core_info()
    NC, NS = info.num_cores, info.num_subcores
    NW = NC * NS
    b_per_w = B_sc // NW
    n_ch = b_per_w // _CHUNK
    mesh = plsc.VectorSubcoreMesh(core_axis_name="c", subcore_axis_name="s")

    @functools.partial(
        pl.kernel,
        mesh=mesh,
        out_type=jax.ShapeDtypeStruct((B_sc, D), jnp.float32),
        scratch_types=[
            pltpu.VMEM((n_ch, _CHUNK), jnp.int32),
            pltpu.VMEM((n_ch, _CHUNK, D), jnp.float32),
            pltpu.SemaphoreType.DMA,
            pltpu.SemaphoreType.DMA,
        ],
    )
    def k(table_hbm, idx_hbm, out_hbm, idx_v, rows_v, gsem, osem):
        wid = lax.axis_index("s") * NC + lax.axis_index("c")
        base = wid * b_per_w
        pltpu.sync_copy(idx_hbm.at[wid], idx_v)
        gathers = []
        for j in range(n_ch):
            gathers.append(
                pltpu.async_copy(table_hbm.at[idx_v.at[j]], rows_v.at[j], gsem)
            )
        outs = []
        for j in range(n_ch):
            gathers[j].wait()
            outs.append(
                pltpu.async_copy(
                    rows_v.at[j], out_hbm.at[pl.ds(base + j * _CHUNK, _CHUNK)], osem
                )
            )
        for o in outs:
            o.wait()

    return k, NW, n_ch


@functools.cache
def _build_tc(B_tc, B, D):
    grid = B_tc // _M_BLK

    def body(idx_ref, tab_ref, out_ref):
        idx = idx_ref[0]
        iota_k = lax.broadcasted_iota(jnp.int32, (_KPAD, _M_BLK), 0)
        oht = (iota_k == idx).astype(jnp.bfloat16)
        res = lax.dot_general(
            oht, tab_ref[...],
            (((0,), (0,)), ((), ())),
            preferred_element_type=jnp.float32,
        )
        out_ref[...] = res[:, :D] + res[:, D:]

    return pl.pallas_call(
        body,
        grid=(grid,),
        in_specs=[
            pl.BlockSpec((1, 1, _M_BLK), lambda i: (i, 0, 0)),
            pl.BlockSpec((_KPAD, 2 * D), lambda i: (0, 0)),
        ],
        out_specs=pl.BlockSpec((_M_BLK, D), lambda i: (i, 0)),
        out_shape=jax.ShapeDtypeStruct((B, D), jnp.float32),
    )


def _split_hi_lo(table):
    bits = lax.bitcast_convert_type(table, jnp.uint32)
    hi = lax.bitcast_convert_type((bits >> 16).astype(jnp.uint16), jnp.bfloat16)
    lo = (table - lax.bitcast_convert_type(bits & jnp.uint32(0xFFFF0000),
                                           jnp.float32)).astype(jnp.bfloat16)
    return hi, lo


def kernel(class_number, embedding):
    V, D = embedding.shape
    B = class_number.shape[0]
    B_tc = B - _B_SC
    idx = class_number.astype(jnp.int32)

    sc_k, NW, n_ch = _build_sc(V, D, _B_SC)
    sc_part = sc_k(embedding, idx[B_tc:].reshape(NW, n_ch, _CHUNK))

    hi, lo = _split_hi_lo(embedding)
    tab2 = jnp.pad(jnp.concatenate([hi, lo], axis=1), ((0, _KPAD - V), (0, 0)))
    tc_full = _build_tc(B_tc, B, D)(idx[:B_tc].reshape(B_tc // _M_BLK, 1, _M_BLK),
                                    tab2)

    return lax.dynamic_update_slice(tc_full, sc_part, (B_tc, 0))


# speedup vs baseline: 1.3081x; 1.3023x over previous
import functools

import jax
import jax.numpy as jnp
from jax import lax
from jax.experimental import pallas as pl
from jax.experimental.pallas import tpu as pltpu
from jax.experimental.pallas import tpu_sc as plsc

_CHUNK = 128


@functools.cache
def _build(V, D, B):
    info = plsc.get_sparse_core_info()
    NC, NS = info.num_cores, info.num_subcores
    NW = NC * NS
    b_per_w = B // NW
    n_ch = b_per_w // _CHUNK
    mesh = plsc.VectorSubcoreMesh(core_axis_name="c", subcore_axis_name="s")

    @functools.partial(
        pl.kernel,
        mesh=mesh,
        out_type=jax.ShapeDtypeStruct((B, D), jnp.float32),
        scratch_types=[
            pltpu.VMEM_SHARED((V, D), jnp.float32),
            pltpu.VMEM((n_ch, _CHUNK), jnp.int32),
            pltpu.VMEM((n_ch, _CHUNK, D), jnp.float32),
            pltpu.SemaphoreType.DMA,
            pltpu.SemaphoreType.DMA,
            pltpu.SemaphoreType.DMA,
        ],
    )
    def k(table_hbm, idx_hbm, out_hbm, tab_s, idx_v, rows_v, tsem, gsem, osem):
        c = lax.axis_index("c")
        s = lax.axis_index("s")
        wid = s * NC + c

        @pl.when(s == 0)
        def _stage_table():
            pltpu.async_copy(table_hbm, tab_s, tsem).wait()

        pltpu.sync_copy(idx_hbm.at[wid], idx_v)
        plsc.subcore_barrier()

        gathers = []
        for j in range(n_ch):
            gathers.append(
                pltpu.async_copy(tab_s.at[idx_v.at[j]], rows_v.at[j], gsem)
            )
        outs = []
        for j in range(n_ch):
            gathers[j].wait()
            outs.append(
                pltpu.async_copy(
                    rows_v.at[j],
                    out_hbm.at[pl.ds(wid * b_per_w + j * _CHUNK, _CHUNK)],
                    osem,
                )
            )
        for o in outs:
            o.wait()

    return k, NW, n_ch


def kernel(class_number, embedding):
    V, D = embedding.shape
    B = class_number.shape[0]
    k, NW, n_ch = _build(V, D, B)
    idx = class_number.astype(jnp.int32).reshape(NW, n_ch, _CHUNK)
    return k(embedding, idx)


# speedup vs baseline: 1.3106x; 1.0019x over previous
import functools

import jax
import jax.numpy as jnp
from jax import lax
from jax.experimental import pallas as pl
from jax.experimental.pallas import tpu as pltpu
from jax.experimental.pallas import tpu_sc as plsc

_CHUNK = 128


@functools.cache
def _build(V, D, B):
    info = plsc.get_sparse_core_info()
    NC, NS = info.num_cores, info.num_subcores
    NW = NC * NS
    b_per_w = B // NW
    n_ch = b_per_w // _CHUNK
    stage_rows = 128
    mesh = plsc.VectorSubcoreMesh(core_axis_name="c", subcore_axis_name="s")

    @functools.partial(
        pl.kernel,
        mesh=mesh,
        out_type=jax.ShapeDtypeStruct((B, D), jnp.float32),
        scratch_types=[
            pltpu.VMEM_SHARED((V, D), jnp.float32),
            pltpu.VMEM((n_ch, _CHUNK), jnp.int32),
            pltpu.VMEM((n_ch, _CHUNK, D), jnp.float32),
            pltpu.SemaphoreType.DMA,
            pltpu.SemaphoreType.DMA,
            pltpu.SemaphoreType.DMA,
        ],
    )
    def k(table_hbm, idx_hbm, out_hbm, tab_s, idx_v, rows_v, tsem, gsem, osem):
        c = lax.axis_index("c")
        s = lax.axis_index("s")
        wid = s * NC + c

        n_stage = (V + stage_rows - 1) // stage_rows
        for i in range(n_stage):
            r0 = i * stage_rows
            nrows = min(stage_rows, V - r0)

            @pl.when(s == i)
            def _stage_table(r0=r0, nrows=nrows):
                pltpu.async_copy(
                    table_hbm.at[pl.ds(r0, nrows)],
                    tab_s.at[pl.ds(r0, nrows)],
                    tsem,
                ).wait()

        pltpu.sync_copy(idx_hbm.at[wid], idx_v)
        plsc.subcore_barrier()

        gathers = []
        for j in range(n_ch):
            gathers.append(
                pltpu.async_copy(tab_s.at[idx_v.at[j]], rows_v.at[j], gsem)
            )
        outs = []
        for j in range(n_ch):
            gathers[j].wait()
            outs.append(
                pltpu.async_copy(
                    rows_v.at[j],
                    out_hbm.at[pl.ds(wid * b_per_w + j * _CHUNK, _CHUNK)],
                    osem,
                )
            )
        for o in outs:
            o.wait()

    return k, NW, n_ch


def kernel(class_number, embedding):
    V, D = embedding.shape
    B = class_number.shape[0]
    k, NW, n_ch = _build(V, D, B)
    idx = class_number.astype(jnp.int32).reshape(NW, n_ch, _CHUNK)
    return k(embedding, idx)
